# baseline (speedup 1.0000x reference)
import jax
import jax.numpy as jnp
from jax import lax
from jax.experimental import pallas as pl
from jax.experimental.pallas import tpu as pltpu
from jax.experimental.pallas import tpu_sc as plsc

_B, _N, _E, _K, _D = 1, 10000, 160000, 4, 128

_NC, _NS, _L = 2, 16, 16
_NW = _NC * _NS
_EPT = _E // _NW
_EPT_PAD = ((_EPT + _L - 1) // _L) * _L
_C = 64
_NCH = -(-_EPT // _C)


def _wid():
    return lax.axis_index("s") * _NC + lax.axis_index("c")


def _w_body(x_hbm, y_hbm, z_hbm, src_hbm, dst_hbm, w_hbm, xv, yv, zv, sv, dv, wv):
    wid = _wid()
    base = wid * _EPT
    pltpu.sync_copy(x_hbm, xv)
    pltpu.sync_copy(y_hbm, yv)
    pltpu.sync_copy(z_hbm, zv)
    zeros = jnp.zeros((_L,), jnp.int32)
    sv[pl.ds(_EPT_PAD - _L, _L)] = zeros
    dv[pl.ds(_EPT_PAD - _L, _L)] = zeros
    pltpu.sync_copy(src_hbm.at[pl.ds(base, _EPT)], sv.at[pl.ds(0, _EPT)])
    pltpu.sync_copy(dst_hbm.at[pl.ds(base, _EPT)], dv.at[pl.ds(0, _EPT)])

    inf = jnp.float32(jnp.inf)
    zero = jnp.float32(0.0)

    def step(i, carry):
        off = i * _L
        s16 = sv[pl.ds(off, _L)]
        d16 = dv[pl.ds(off, _L)]
        dx = plsc.load_gather(xv, [s16]) - plsc.load_gather(xv, [d16])
        dy = plsc.load_gather(yv, [s16]) - plsc.load_gather(yv, [d16])
        dz = plsc.load_gather(zv, [s16]) - plsc.load_gather(zv, [d16])
        d2 = dx * dx + dy * dy + dz * dz
        w = jnp.float32(1.0) / d2
        wv[pl.ds(off, _L)] = jnp.where(w == inf, zero, w)
        return carry

    lax.fori_loop(0, _EPT_PAD // _L, step, 0)
    pltpu.sync_copy(wv.at[pl.ds(0, _EPT)], w_hbm.at[pl.ds(base, _EPT)])


_w_kernel = pl.kernel(
    _w_body,
    out_type=jax.ShapeDtypeStruct((_E,), jnp.float32),
    mesh=plsc.VectorSubcoreMesh(core_axis_name="c", subcore_axis_name="s"),
    compiler_params=pltpu.CompilerParams(needs_layout_passes=False),
    scratch_types=[
        pltpu.VMEM((_N,), jnp.float32),
        pltpu.VMEM((_N,), jnp.float32),
        pltpu.VMEM((_N,), jnp.float32),
        pltpu.VMEM((_EPT_PAD,), jnp.int32),
        pltpu.VMEM((_EPT_PAD,), jnp.int32),
        pltpu.VMEM((_EPT_PAD,), jnp.float32),
    ],
)


def _msg_body(r_hbm, w_hbm, nbr_hbm, out_hbm, idxf_v, idx_v, rows_v, wk_v,
              out_v, isem, gsem, osem):
    wid = _wid()
    ebase = wid * _EPT

    def _coff(ch):
        return jnp.minimum(ch * _C, _EPT - _C)

    def idx_copy(ch, b):
        for h in range(2):
            pltpu.async_copy(
                nbr_hbm.at[pl.ds((ebase + _coff(ch)) * _K + 128 * h, 128)],
                idxf_v.at[b, h], isem.at[b])

    def wait_idx(ch, b):
        for h in range(2):
            pltpu.make_async_copy(
                nbr_hbm.at[pl.ds((ebase + _coff(ch)) * _K + 128 * h, 128)],
                idxf_v.at[b, h], isem.at[b]).wait()

    def repack_idx(b):
        lanes = jnp.arange(_L, dtype=jnp.int32) * _K
        for k in range(_K):
            for g in range(_C // _L):
                local = _K * _L * g - 128 * (g // 2) + k
                vec = plsc.load_gather(idxf_v.at[b, g // 2], [lanes + local])
                idx_v[b, k // 2, pl.ds((k % 2) * _C + _L * g, _L)] = vec

    def issue_gathers(b):
        for p in range(2):
            pltpu.async_copy(r_hbm.at[idx_v.at[b, p]], rows_v.at[b, p],
                             gsem.at[b])
            pltpu.async_copy(w_hbm.at[idx_v.at[b, p]], wk_v.at[b, p],
                             gsem.at[b])

    def wait_gathers(b):
        for p in range(2):
            pltpu.make_async_copy(r_hbm.at[idx_v.at[b, p]],
                                  rows_v.at[b, p], gsem.at[b]).wait()
            pltpu.make_async_copy(w_hbm.at[idx_v.at[b, p]],
                                  wk_v.at[b, p], gsem.at[b]).wait()

    def out_write(ch, b):
        pltpu.async_copy(out_v.at[b],
                         out_hbm.at[pl.ds(ebase + _coff(ch), _C)],
                         osem.at[b])

    def wait_out(ch, b):
        pltpu.make_async_copy(out_v.at[b],
                              out_hbm.at[pl.ds(ebase + _coff(ch), _C)],
                              osem.at[b]).wait()

    def compute(b):
        def edge(i, ecarry):
            idxi = jnp.full((_L,), i, dtype=jnp.int32)
            idxi2 = idxi + _C
            i2 = i + _C
            w0 = plsc.load_gather(wk_v.at[b, 0], [idxi])
            w1 = plsc.load_gather(wk_v.at[b, 0], [idxi2])
            w2 = plsc.load_gather(wk_v.at[b, 1], [idxi])
            w3 = plsc.load_gather(wk_v.at[b, 1], [idxi2])
            for j in range(_D // _L):
                sl = pl.ds(j * _L, _L)
                a01 = w0 * rows_v[b, 0, i, sl] + w1 * rows_v[b, 0, i2, sl]
                a23 = w2 * rows_v[b, 1, i, sl] + w3 * rows_v[b, 1, i2, sl]
                out_v[b, i, sl] = a01 + a23
            return ecarry

        lax.fori_loop(0, _C, edge, 0, unroll=4)

    idx_copy(0, 0)
    idx_copy(1, 1)
    wait_idx(0, 0)
    repack_idx(0)
    issue_gathers(0)

    def pair(it, carry):
        for b in range(2):
            ch = it * 2 + b
            wait_gathers(b)

            @pl.when(ch + 2 < _NCH)
            def _():
                idx_copy(ch + 2, b)

            @pl.when(ch + 1 < _NCH)
            def _():
                wait_idx(ch + 1, 1 - b)
                repack_idx(1 - b)
                issue_gathers(1 - b)

            @pl.when(ch >= 2)
            def _():
                wait_out(ch - 2, b)

            compute(b)
            out_write(ch, b)
        return carry

    lax.fori_loop(0, _NCH // 2, pair, 0)

    if _NCH % 2 == 1:
        last = _NCH - 1
        wait_gathers(0)
        wait_out(last - 2, 0)
        compute(0)
        out_write(last, 0)
        wait_out(last - 1, 1)
        wait_out(last, 0)
    else:
        wait_out(_NCH - 2, 0)
        wait_out(_NCH - 1, 1)


_msg_kernel = pl.kernel(
    _msg_body,
    out_type=jax.ShapeDtypeStruct((_E, _D), jnp.float32),
    mesh=plsc.VectorSubcoreMesh(core_axis_name="c", subcore_axis_name="s"),
    compiler_params=pltpu.CompilerParams(needs_layout_passes=False),
    scratch_types=[
        pltpu.VMEM((2, 2, _K * _C // 2), jnp.int32),
        pltpu.VMEM((2, 2, 2 * _C), jnp.int32),
        pltpu.VMEM((2, 2, 2 * _C, _D), jnp.float32),
        pltpu.VMEM((2, 2, 2 * _C), jnp.float32),
        pltpu.VMEM((2, _C, _D), jnp.float32),
        pltpu.SemaphoreType.DMA((2,)),
        pltpu.SemaphoreType.DMA((2,)),
        pltpu.SemaphoreType.DMA((2,)),
    ],
)


def kernel(bond_representations, bond_pairs, bond_neighbors, xyz):
    r = bond_representations[0]
    src = bond_pairs[0, :, 0]
    dst = bond_pairs[0, :, 1]
    x = xyz[0, :, 0]
    y = xyz[0, :, 1]
    z = xyz[0, :, 2]
    nbr = bond_neighbors[0].reshape(_E * _K)
    w = _w_kernel(x, y, z, src, dst)
    out = _msg_kernel(r, w, nbr)
    return out.reshape(1, _B, _E, _D)


# speedup vs baseline: 1.0009x; 1.0009x over previous
import jax
import jax.numpy as jnp
from jax import lax
from jax.experimental import pallas as pl
from jax.experimental.pallas import tpu as pltpu
from jax.experimental.pallas import tpu_sc as plsc

_B, _N, _E, _K, _D = 1, 10000, 160000, 4, 128

_NC, _NS, _L = 2, 16, 16
_NW = _NC * _NS
_EPT = _E // _NW
_EPT_PAD = ((_EPT + _L - 1) // _L) * _L
_C = 64
_NCH = -(-_EPT // _C)


def _wid():
    return lax.axis_index("s") * _NC + lax.axis_index("c")


def _w_body(x_hbm, y_hbm, z_hbm, src_hbm, dst_hbm, w_hbm, xv, yv, zv, sv, dv, wv):
    wid = _wid()
    base = wid * _EPT
    pltpu.sync_copy(x_hbm, xv)
    pltpu.sync_copy(y_hbm, yv)
    pltpu.sync_copy(z_hbm, zv)
    zeros = jnp.zeros((_L,), jnp.int32)
    sv[pl.ds(_EPT_PAD - _L, _L)] = zeros
    dv[pl.ds(_EPT_PAD - _L, _L)] = zeros
    pltpu.sync_copy(src_hbm.at[pl.ds(base, _EPT)], sv.at[pl.ds(0, _EPT)])
    pltpu.sync_copy(dst_hbm.at[pl.ds(base, _EPT)], dv.at[pl.ds(0, _EPT)])

    inf = jnp.float32(jnp.inf)
    zero = jnp.float32(0.0)

    def step(i, carry):
        off = i * _L
        s16 = sv[pl.ds(off, _L)]
        d16 = dv[pl.ds(off, _L)]
        dx = plsc.load_gather(xv, [s16]) - plsc.load_gather(xv, [d16])
        dy = plsc.load_gather(yv, [s16]) - plsc.load_gather(yv, [d16])
        dz = plsc.load_gather(zv, [s16]) - plsc.load_gather(zv, [d16])
        d2 = dx * dx + dy * dy + dz * dz
        w = jnp.float32(1.0) / d2
        wv[pl.ds(off, _L)] = jnp.where(w == inf, zero, w)
        return carry

    lax.fori_loop(0, _EPT_PAD // _L, step, 0)
    pltpu.sync_copy(wv.at[pl.ds(0, _EPT)], w_hbm.at[pl.ds(base, _EPT)])


_w_kernel = pl.kernel(
    _w_body,
    out_type=jax.ShapeDtypeStruct((_E,), jnp.float32),
    mesh=plsc.VectorSubcoreMesh(core_axis_name="c", subcore_axis_name="s", num_cores=_NC, num_subcores=_NS),
    compiler_params=pltpu.CompilerParams(needs_layout_passes=False),
    scratch_types=[
        pltpu.VMEM((_N,), jnp.float32),
        pltpu.VMEM((_N,), jnp.float32),
        pltpu.VMEM((_N,), jnp.float32),
        pltpu.VMEM((_EPT_PAD,), jnp.int32),
        pltpu.VMEM((_EPT_PAD,), jnp.int32),
        pltpu.VMEM((_EPT_PAD,), jnp.float32),
    ],
)


def _msg_body(r_hbm, w_hbm, nbr_hbm, out_hbm, idxf_v, idx_v, rows_v, wk_v,
              out_v, isem, gsem, osem):
    wid = _wid()
    ebase = wid * _EPT

    def _coff(ch):
        return jnp.minimum(ch * _C, _EPT - _C)

    def idx_copy(ch, b):
        for h in range(2):
            pltpu.async_copy(
                nbr_hbm.at[pl.ds((ebase + _coff(ch)) * _K + 128 * h, 128)],
                idxf_v.at[b, h], isem.at[b])

    def wait_idx(ch, b):
        for h in range(2):
            pltpu.make_async_copy(
                nbr_hbm.at[pl.ds((ebase + _coff(ch)) * _K + 128 * h, 128)],
                idxf_v.at[b, h], isem.at[b]).wait()

    def repack_idx(b):
        lanes = jnp.arange(_L, dtype=jnp.int32) * _K
        for k in range(_K):
            for g in range(_C // _L):
                local = _K * _L * g - 128 * (g // 2) + k
                vec = plsc.load_gather(idxf_v.at[b, g // 2], [lanes + local])
                idx_v[b, k // 2, pl.ds((k % 2) * _C + _L * g, _L)] = vec

    def issue_gathers(b):
        for p in range(2):
            pltpu.async_copy(r_hbm.at[idx_v.at[b, p]], rows_v.at[b, p],
                             gsem.at[b])
            pltpu.async_copy(w_hbm.at[idx_v.at[b, p]], wk_v.at[b, p],
                             gsem.at[b])

    def wait_gathers(b):
        for p in range(2):
            pltpu.make_async_copy(r_hbm.at[idx_v.at[b, p]],
                                  rows_v.at[b, p], gsem.at[b]).wait()
            pltpu.make_async_copy(w_hbm.at[idx_v.at[b, p]],
                                  wk_v.at[b, p], gsem.at[b]).wait()

    def out_write(ch, b):
        pltpu.async_copy(out_v.at[b],
                         out_hbm.at[pl.ds(ebase + _coff(ch), _C)],
                         osem.at[b])

    def wait_out(ch, b):
        pltpu.make_async_copy(out_v.at[b],
                              out_hbm.at[pl.ds(ebase + _coff(ch), _C)],
                              osem.at[b]).wait()

    def compute(b):
        def edge(i, ecarry):
            idxi = jnp.full((_L,), i, dtype=jnp.int32)
            idxi2 = idxi + _C
            i2 = i + _C
            w0 = plsc.load_gather(wk_v.at[b, 0], [idxi])
            w1 = plsc.load_gather(wk_v.at[b, 0], [idxi2])
            w2 = plsc.load_gather(wk_v.at[b, 1], [idxi])
            w3 = plsc.load_gather(wk_v.at[b, 1], [idxi2])
            for j in range(_D // _L):
                sl = pl.ds(j * _L, _L)
                a01 = w0 * rows_v[b, 0, i, sl] + w1 * rows_v[b, 0, i2, sl]
                a23 = w2 * rows_v[b, 1, i, sl] + w3 * rows_v[b, 1, i2, sl]
                out_v[b, i, sl] = a01 + a23
            return ecarry

        lax.fori_loop(0, _C, edge, 0, unroll=4)

    idx_copy(0, 0)
    idx_copy(1, 1)
    wait_idx(0, 0)
    repack_idx(0)
    issue_gathers(0)

    def pair(it, carry):
        for b in range(2):
            ch = it * 2 + b
            wait_gathers(b)

            @pl.when(ch + 2 < _NCH)
            def _():
                idx_copy(ch + 2, b)

            @pl.when(ch + 1 < _NCH)
            def _():
                wait_idx(ch + 1, 1 - b)
                repack_idx(1 - b)
                issue_gathers(1 - b)

            @pl.when(ch >= 2)
            def _():
                wait_out(ch - 2, b)

            compute(b)
            out_write(ch, b)
        return carry

    lax.fori_loop(0, _NCH // 2, pair, 0)

    if _NCH % 2 == 1:
        last = _NCH - 1
        wait_gathers(0)
        wait_out(last - 2, 0)
        compute(0)
        out_write(last, 0)
        wait_out(last - 1, 1)
        wait_out(last, 0)
    else:
        wait_out(_NCH - 2, 0)
        wait_out(_NCH - 1, 1)


_msg_kernel = pl.kernel(
    _msg_body,
    out_type=jax.ShapeDtypeStruct((_E, _D), jnp.float32),
    mesh=plsc.VectorSubcoreMesh(core_axis_name="c", subcore_axis_name="s", num_cores=_NC, num_subcores=_NS),
    compiler_params=pltpu.CompilerParams(needs_layout_passes=False),
    scratch_types=[
        pltpu.VMEM((2, 2, _K * _C // 2), jnp.int32),
        pltpu.VMEM((2, 2, 2 * _C), jnp.int32),
        pltpu.VMEM((2, 2, 2 * _C, _D), jnp.float32),
        pltpu.VMEM((2, 2, 2 * _C), jnp.float32),
        pltpu.VMEM((2, _C, _D), jnp.float32),
        pltpu.SemaphoreType.DMA((2,)),
        pltpu.SemaphoreType.DMA((2,)),
        pltpu.SemaphoreType.DMA((2,)),
    ],
)


def kernel(bond_representations, bond_pairs, bond_neighbors, xyz):
    r = bond_representations[0]
    src = bond_pairs[0, :, 0]
    dst = bond_pairs[0, :, 1]
    x = xyz[0, :, 0]
    y = xyz[0, :, 1]
    z = xyz[0, :, 2]
    nbr = bond_neighbors[0].reshape(_E * _K)
    w = _w_kernel(x, y, z, src, dst)
    out = _msg_kernel(r, w, nbr)
    return out.reshape(1, _B, _E, _D)
